# baseline (speedup 1.0000x reference)
import functools

import jax
import jax.numpy as jnp
from jax import lax
from jax.experimental import pallas as pl
from jax.experimental.pallas import tpu as pltpu
from jax.experimental.pallas import tpu_sc as plsc

N = 10000
NPAD = 10240
E = 320000
D = 128
H = 64
G = 64
NC = 2
NS = 16
NW = NC * NS
CH = 128
NCHUNK = 80
EPT = NCHUNK * CH
EPAD = NW * EPT
RPT = NPAD // NS
NSEC = 4
SEC = NCHUNK // NSEC


def _make_agg(dh, nbuf):
  grp = dh // 16
  assert SEC % nbuf == 0
  mesh = plsc.VectorSubcoreMesh(core_axis_name="c", subcore_axis_name="s")

  @functools.partial(
      pl.kernel,
      mesh=mesh,
      compiler_params=pltpu.CompilerParams(use_tc_tiling_on_sc=False),
      out_type=[
          jax.ShapeDtypeStruct((NPAD, dh), jnp.float32),
          jax.ShapeDtypeStruct((NPAD, dh), jnp.float32),
      ],
      scratch_types=[
          pltpu.VMEM((SEC, CH), jnp.int32),
          pltpu.VMEM((SEC, CH), jnp.int32),
          [pltpu.VMEM((CH, dh), jnp.float32) for _ in range(nbuf)],
          pltpu.VMEM_SHARED((NPAD, dh), jnp.float32),
          [pltpu.SemaphoreType.DMA for _ in range(nbuf)],
      ],
  )
  def agg(h_hbm, src_hbm, dst_hbm, out0, out1, sidx, didx, bufs, acc, gsems):
    cid = lax.axis_index("c")
    sid = lax.axis_index("s")
    wid = cid * NS + sid

    zeros16 = jnp.zeros((16,), jnp.float32)

    def zbody(k, c):
      bufs[0][k // grp, pl.ds((k % grp) * 16, 16)] = zeros16
      return c

    lax.fori_loop(0, CH * grp, zbody, 0)
    rbase = sid * RPT
    for k in range(RPT // CH):
      pltpu.sync_copy(bufs[0], acc.at[pl.ds(rbase + k * CH, CH)])
    plsc.subcore_barrier()

    def group(jj, c):
      j0 = jj * nbuf
      gcps = [
          pltpu.async_copy(h_hbm.at[sidx.at[j0 + b]], bufs[b], gsems[b])
          for b in range(nbuf)
      ]
      for b in range(nbuf):
        gcps[b].wait()
        pltpu.sync_copy(bufs[b], acc.at[didx.at[j0 + b]], add=True)
      return c

    for s in range(NSEC):
      pltpu.sync_copy(src_hbm.at[wid * NSEC + s], sidx)
      pltpu.sync_copy(dst_hbm.at[wid * NSEC + s], didx)
      lax.fori_loop(0, SEC // nbuf, group, 0)
    plsc.subcore_barrier()

    @pl.when(cid == 0)
    def _():
      pltpu.sync_copy(acc.at[pl.ds(rbase, RPT)], out0.at[pl.ds(rbase, RPT)])

    @pl.when(cid == 1)
    def _():
      pltpu.sync_copy(acc.at[pl.ds(rbase, RPT)], out1.at[pl.ds(rbase, RPT)])

  return agg


_AGG128 = _make_agg(D, 2)
_AGG64 = _make_agg(H, 5)

_R = 1024
_NB = NPAD // _R


def _mlp_math(h_ref, p0_ref, p1_ref, w1_ref, aux_ref, w2_ref):
  aux = aux_ref[...]
  b1 = aux[0:1]
  gg = aux[1:2]
  bt = aux[2:3]
  mm = aux[3:4]
  vv = aux[4:5]
  b2 = aux[5:6]
  hin = h_ref[...] + p0_ref[...] + p1_ref[...]
  z = jnp.dot(hin, w1_ref[...], preferred_element_type=jnp.float32) + b1
  sc = gg * lax.rsqrt(vv + 1e-5)
  z = jnp.maximum(z * sc + (bt - mm * sc), 0.0)
  return jnp.maximum(
      jnp.dot(z, w2_ref[...], preferred_element_type=jnp.float32) + b2, 0.0)


def _mlp_block(h_ref, p0_ref, p1_ref, w1_ref, aux_ref, w2_ref, o_ref):
  o_ref[...] = _mlp_math(h_ref, p0_ref, p1_ref, w1_ref, aux_ref, w2_ref)


def _mlp_layer(h, p0, p1, w1, aux, w2):
  din = h.shape[1]
  return pl.pallas_call(
      _mlp_block,
      grid=(_NB,),
      in_specs=[
          pl.BlockSpec((_R, din), lambda i: (i, 0)),
          pl.BlockSpec((_R, din), lambda i: (i, 0)),
          pl.BlockSpec((_R, din), lambda i: (i, 0)),
          pl.BlockSpec((din, H), lambda i: (0, 0)),
          pl.BlockSpec((8, H), lambda i: (0, 0)),
          pl.BlockSpec((H, H), lambda i: (0, 0)),
      ],
      out_specs=pl.BlockSpec((_R, H), lambda i: (i, 0)),
      out_shape=jax.ShapeDtypeStruct((NPAD, H), jnp.float32),
  )(h, p0, p1, w1, aux, w2)


def _final_block(h_ref, p0_ref, p1_ref, w1_ref, aux_ref, w2_ref, bc_ref,
                 l1w_ref, l2w_ref, hb_ref, o_ref, pacc):
  i = pl.program_id(0)
  h3 = _mlp_math(h_ref, p0_ref, p1_ref, w1_ref, aux_ref, w2_ref)
  bc = bc_ref[...]
  seg = lax.broadcasted_iota(jnp.int32, (_R, G), 1)
  oh = (bc == seg).astype(jnp.float32)
  part = lax.dot_general(oh, h3, (((0,), (0,)), ((), ())),
                         preferred_element_type=jnp.float32)

  @pl.when(i == 0)
  def _():
    pacc[...] = part

  @pl.when(i > 0)
  def _():
    pacc[...] += part

  @pl.when(i == _NB - 1)
  def _():
    pooled = pacc[...]
    l1 = l1w_ref[...]
    leff = l1[0:H] + l1[H:2 * H] + l1[2 * H:3 * H]
    hb = hb_ref[...]
    h1 = jnp.maximum(
        jnp.dot(pooled, leff, preferred_element_type=jnp.float32) + hb[0:1, :],
        0.0)
    logits = jnp.dot(h1, l2w_ref[...],
                     preferred_element_type=jnp.float32) + hb[1:2, 0:2]
    mx = jnp.max(logits, axis=1, keepdims=True)
    s = jnp.sum(jnp.exp(logits - mx), axis=1, keepdims=True)
    o_ref[...] = logits - mx - jnp.log(s)


def _final_layer(h, p0, p1, w1, aux, w2, bcol, l1w, l2w, hb):
  return pl.pallas_call(
      _final_block,
      grid=(_NB,),
      in_specs=[
          pl.BlockSpec((_R, H), lambda i: (i, 0)),
          pl.BlockSpec((_R, H), lambda i: (i, 0)),
          pl.BlockSpec((_R, H), lambda i: (i, 0)),
          pl.BlockSpec((H, H), lambda i: (0, 0)),
          pl.BlockSpec((8, H), lambda i: (0, 0)),
          pl.BlockSpec((H, H), lambda i: (0, 0)),
          pl.BlockSpec((_R, 1), lambda i: (i, 0)),
          pl.BlockSpec((3 * H, H), lambda i: (0, 0)),
          pl.BlockSpec((H, 2), lambda i: (0, 0)),
          pl.BlockSpec((8, H), lambda i: (0, 0)),
      ],
      out_specs=pl.BlockSpec((G, 2), lambda i: (0, 0)),
      out_shape=jax.ShapeDtypeStruct((G, 2), jnp.float32),
      scratch_shapes=[pltpu.VMEM((G, H), jnp.float32)],
  )(h, p0, p1, w1, aux, w2, bcol, l1w, l2w, hb)


def _aux_stack(b1, g, bt, m, v, b2):
  return jnp.concatenate(
      [b1[None], g[None], bt[None], m[None], v[None], b2[None],
       jnp.zeros((2, H), jnp.float32)], axis=0)


def kernel(x, edge_index, batch, W1_0, b1_0, g_0, bt_0, m_0, v_0, W2_0, b2_0,
           W1_1, b1_1, g_1, bt_1, m_1, v_1, W2_1, b2_1,
           W1_2, b1_2, g_2, bt_2, m_2, v_2, W2_2, b2_2,
           lin1_W, lin1_b, lin2_W, lin2_b):
  pe = EPAD - E
  srcp = jnp.concatenate(
      [edge_index[0], jnp.zeros((pe,), jnp.int32)]).reshape(NW * NSEC, SEC, CH)
  dstp = jnp.concatenate(
      [edge_index[1],
       N + (jnp.arange(pe, dtype=jnp.int32) % (NPAD - N))]).reshape(
           NW * NSEC, SEC, CH)
  xp = jnp.concatenate([x, jnp.zeros((NPAD - N, D), jnp.float32)], axis=0)
  bcol = jnp.concatenate(
      [batch, jnp.full((NPAD - N,), G, jnp.int32)]).reshape(NPAD, 1)
  aux0 = _aux_stack(b1_0, g_0, bt_0, m_0, v_0, b2_0)
  aux1 = _aux_stack(b1_1, g_1, bt_1, m_1, v_1, b2_1)
  aux2 = _aux_stack(b1_2, g_2, bt_2, m_2, v_2, b2_2)
  hb = jnp.zeros((8, H), jnp.float32).at[0].set(lin1_b).at[1, :2].set(lin2_b)

  p0, p1 = _AGG128(xp, srcp, dstp)
  h1 = _mlp_layer(xp, p0, p1, W1_0, aux0, W2_0)
  p0, p1 = _AGG64(h1, srcp, dstp)
  h2 = _mlp_layer(h1, p0, p1, W1_1, aux1, W2_1)
  p0, p1 = _AGG64(h2, srcp, dstp)
  return _final_layer(h2, p0, p1, W1_2, aux2, W2_2, bcol, lin1_W, lin2_W, hb)


# speedup vs baseline: 2.9762x; 2.9762x over previous
import functools

import jax
import jax.numpy as jnp
from jax import lax
from jax.experimental import pallas as pl
from jax.experimental.pallas import tpu as pltpu
from jax.experimental.pallas import tpu_sc as plsc

N = 10000
NPAD = 10240
E = 320000
D = 128
H = 64
G = 64
NC = 2
NS = 16
NW = NC * NS
CH = 128
NCHUNK = 80
EPT = NCHUNK * CH
EPAD = NW * EPT
RPT = NPAD // NS
NSEC = 4
SEC = NCHUNK // NSEC


def _make_agg(dh, nbuf):
  grp = dh // 16
  assert SEC % nbuf == 0
  mesh = plsc.VectorSubcoreMesh(core_axis_name="c", subcore_axis_name="s")

  @functools.partial(
      pl.kernel,
      mesh=mesh,
      compiler_params=pltpu.CompilerParams(use_tc_tiling_on_sc=False),
      out_type=[
          jax.ShapeDtypeStruct((NPAD, dh), jnp.float32),
          jax.ShapeDtypeStruct((NPAD, dh), jnp.float32),
      ],
      scratch_types=[
          pltpu.VMEM((SEC, CH), jnp.int32),
          pltpu.VMEM((SEC, CH), jnp.int32),
          [pltpu.VMEM((CH, dh), jnp.float32) for _ in range(nbuf)],
          pltpu.VMEM_SHARED((NPAD, dh), jnp.float32),
          [pltpu.SemaphoreType.DMA for _ in range(nbuf)],
      ],
  )
  def agg(h_hbm, src_hbm, dst_hbm, out0, out1, sidx, didx, bufs, acc, gsems):
    cid = lax.axis_index("c")
    sid = lax.axis_index("s")
    wid = cid * NS + sid

    zeros16 = jnp.zeros((16,), jnp.float32)

    def zbody(k, c):
      bufs[0][k // grp, pl.ds((k % grp) * 16, 16)] = zeros16
      return c

    lax.fori_loop(0, CH * grp, zbody, 0)
    rbase = sid * RPT
    for k in range(RPT // CH):
      pltpu.sync_copy(bufs[0], acc.at[pl.ds(rbase + k * CH, CH)])
    plsc.subcore_barrier()

    def group(jj, c):
      j0 = jj * nbuf
      gcps = [
          pltpu.async_copy(h_hbm.at[sidx.at[j0 + b]], bufs[b], gsems[b])
          for b in range(nbuf)
      ]
      for b in range(nbuf):
        gcps[b].wait()
        pltpu.sync_copy(bufs[b], acc.at[didx.at[j0 + b]], add=True)
      return c

    for s in range(NSEC):
      pltpu.sync_copy(src_hbm.at[wid * NSEC + s], sidx)
      pltpu.sync_copy(dst_hbm.at[wid * NSEC + s], didx)
      lax.fori_loop(0, SEC // nbuf, group, 0)
    plsc.subcore_barrier()

    @pl.when(cid == 0)
    def _():
      pltpu.sync_copy(acc.at[pl.ds(rbase, RPT)], out0.at[pl.ds(rbase, RPT)])

    @pl.when(cid == 1)
    def _():
      pltpu.sync_copy(acc.at[pl.ds(rbase, RPT)], out1.at[pl.ds(rbase, RPT)])

  return agg


_AGG128 = _make_agg(D, 2)
_AGG64 = _make_agg(H, 5)

_R = 1024
_NB = NPAD // _R


def _mlp_math(h_ref, p0_ref, p1_ref, w1_ref, aux_ref, w2_ref):
  aux = aux_ref[...]
  b1 = aux[0:1]
  gg = aux[1:2]
  bt = aux[2:3]
  mm = aux[3:4]
  vv = aux[4:5]
  b2 = aux[5:6]
  hin = h_ref[...] + p0_ref[...] + p1_ref[...]
  z = jnp.dot(hin, w1_ref[...], preferred_element_type=jnp.float32) + b1
  sc = gg * lax.rsqrt(vv + 1e-5)
  z = jnp.maximum(z * sc + (bt - mm * sc), 0.0)
  return jnp.maximum(
      jnp.dot(z, w2_ref[...], preferred_element_type=jnp.float32) + b2, 0.0)


def _mlp_block(h_ref, p0_ref, p1_ref, w1_ref, aux_ref, w2_ref, o_ref):
  o_ref[...] = _mlp_math(h_ref, p0_ref, p1_ref, w1_ref, aux_ref, w2_ref)


def _mlp_layer(h, p0, p1, w1, aux, w2):
  din = h.shape[1]
  return pl.pallas_call(
      _mlp_block,
      grid=(_NB,),
      in_specs=[
          pl.BlockSpec((_R, din), lambda i: (i, 0)),
          pl.BlockSpec((_R, din), lambda i: (i, 0)),
          pl.BlockSpec((_R, din), lambda i: (i, 0)),
          pl.BlockSpec((din, H), lambda i: (0, 0)),
          pl.BlockSpec((8, H), lambda i: (0, 0)),
          pl.BlockSpec((H, H), lambda i: (0, 0)),
      ],
      out_specs=pl.BlockSpec((_R, H), lambda i: (i, 0)),
      out_shape=jax.ShapeDtypeStruct((NPAD, H), jnp.float32),
  )(h, p0, p1, w1, aux, w2)


def _final_block(h_ref, p0_ref, p1_ref, w1_ref, aux_ref, w2_ref, bc_ref,
                 l1w_ref, l2w_ref, hb_ref, o_ref, pacc):
  i = pl.program_id(0)
  h3 = _mlp_math(h_ref, p0_ref, p1_ref, w1_ref, aux_ref, w2_ref)
  bc = bc_ref[...]
  seg = lax.broadcasted_iota(jnp.int32, (_R, G), 1)
  oh = (bc == seg).astype(jnp.float32)
  part = lax.dot_general(oh, h3, (((0,), (0,)), ((), ())),
                         preferred_element_type=jnp.float32)

  @pl.when(i == 0)
  def _():
    pacc[...] = part

  @pl.when(i > 0)
  def _():
    pacc[...] += part

  @pl.when(i == _NB - 1)
  def _():
    pooled = pacc[...]
    l1 = l1w_ref[...]
    leff = l1[0:H] + l1[H:2 * H] + l1[2 * H:3 * H]
    hb = hb_ref[...]
    h1 = jnp.maximum(
        jnp.dot(pooled, leff, preferred_element_type=jnp.float32) + hb[0:1, :],
        0.0)
    logits = jnp.dot(h1, l2w_ref[...],
                     preferred_element_type=jnp.float32) + hb[1:2, 0:2]
    mx = jnp.max(logits, axis=1, keepdims=True)
    s = jnp.sum(jnp.exp(logits - mx), axis=1, keepdims=True)
    o_ref[...] = logits - mx - jnp.log(s)


def _final_layer(h, p0, p1, w1, aux, w2, bcol, l1w, l2w, hb):
  return pl.pallas_call(
      _final_block,
      grid=(_NB,),
      in_specs=[
          pl.BlockSpec((_R, H), lambda i: (i, 0)),
          pl.BlockSpec((_R, H), lambda i: (i, 0)),
          pl.BlockSpec((_R, H), lambda i: (i, 0)),
          pl.BlockSpec((H, H), lambda i: (0, 0)),
          pl.BlockSpec((8, H), lambda i: (0, 0)),
          pl.BlockSpec((H, H), lambda i: (0, 0)),
          pl.BlockSpec((_R, 1), lambda i: (i, 0)),
          pl.BlockSpec((3 * H, H), lambda i: (0, 0)),
          pl.BlockSpec((H, 2), lambda i: (0, 0)),
          pl.BlockSpec((8, H), lambda i: (0, 0)),
      ],
      out_specs=pl.BlockSpec((G, 2), lambda i: (0, 0)),
      out_shape=jax.ShapeDtypeStruct((G, 2), jnp.float32),
      scratch_shapes=[pltpu.VMEM((G, H), jnp.float32)],
  )(h, p0, p1, w1, aux, w2, bcol, l1w, l2w, hb)


def _aux_stack(b1, g, bt, m, v, b2):
  return jnp.concatenate(
      [b1[None], g[None], bt[None], m[None], v[None], b2[None],
       jnp.zeros((2, H), jnp.float32)], axis=0)


def kernel(x, edge_index, batch, W1_0, b1_0, g_0, bt_0, m_0, v_0, W2_0, b2_0,
           W1_1, b1_1, g_1, bt_1, m_1, v_1, W2_1, b2_1,
           W1_2, b1_2, g_2, bt_2, m_2, v_2, W2_2, b2_2,
           lin1_W, lin1_b, lin2_W, lin2_b):
  pe = EPAD - E
  srcp = jnp.concatenate(
      [edge_index[0],
       jnp.arange(pe, dtype=jnp.int32) % N]).reshape(NW * NSEC, SEC, CH)
  dstp = jnp.concatenate(
      [edge_index[1],
       N + (jnp.arange(pe, dtype=jnp.int32) % (NPAD - N))]).reshape(
           NW * NSEC, SEC, CH)
  xp = jnp.concatenate([x, jnp.zeros((NPAD - N, D), jnp.float32)], axis=0)
  bcol = jnp.concatenate(
      [batch, jnp.full((NPAD - N,), G, jnp.int32)]).reshape(NPAD, 1)
  aux0 = _aux_stack(b1_0, g_0, bt_0, m_0, v_0, b2_0)
  aux1 = _aux_stack(b1_1, g_1, bt_1, m_1, v_1, b2_1)
  aux2 = _aux_stack(b1_2, g_2, bt_2, m_2, v_2, b2_2)
  hb = jnp.zeros((8, H), jnp.float32).at[0].set(lin1_b).at[1, :2].set(lin2_b)

  p0, p1 = _AGG128(xp, srcp, dstp)
  h1 = _mlp_layer(xp, p0, p1, W1_0, aux0, W2_0)
  p0, p1 = _AGG64(h1, srcp, dstp)
  h2 = _mlp_layer(h1, p0, p1, W1_1, aux1, W2_1)
  p0, p1 = _AGG64(h2, srcp, dstp)
  return _final_layer(h2, p0, p1, W1_2, aux2, W2_2, bcol, lin1_W, lin2_W, hb)


# speedup vs baseline: 3.3731x; 1.1334x over previous
import functools

import jax
import jax.numpy as jnp
from jax import lax
from jax.experimental import pallas as pl
from jax.experimental.pallas import tpu as pltpu
from jax.experimental.pallas import tpu_sc as plsc

N = 10000
NPAD = 10240
E = 320000
D = 128
H = 64
G = 64
NC = 2
NS = 16
NW = NC * NS
CH = 128
NCHUNK = 80
EPT = NCHUNK * CH
EPAD = NW * EPT
RPT = NPAD // NS
NSEC = 4
SEC = NCHUNK // NSEC


def _make_agg(dh, nbuf):
  grp = dh // 16
  assert SEC % nbuf == 0
  mesh = plsc.VectorSubcoreMesh(core_axis_name="c", subcore_axis_name="s")

  @functools.partial(
      pl.kernel,
      mesh=mesh,
      compiler_params=pltpu.CompilerParams(use_tc_tiling_on_sc=False),
      out_type=[
          jax.ShapeDtypeStruct((NPAD, dh), jnp.float32),
          jax.ShapeDtypeStruct((NPAD, dh), jnp.float32),
      ],
      scratch_types=[
          pltpu.VMEM((SEC, CH), jnp.int32),
          pltpu.VMEM((SEC, CH), jnp.int32),
          [pltpu.VMEM((CH, dh), jnp.float32) for _ in range(nbuf)],
          pltpu.VMEM_SHARED((NPAD, dh), jnp.float32),
          [pltpu.SemaphoreType.DMA for _ in range(nbuf)],
      ],
  )
  def agg(h_hbm, src_hbm, dst_hbm, out0, out1, sidx, didx, bufs, acc, gsems):
    cid = lax.axis_index("c")
    sid = lax.axis_index("s")
    wid = cid * NS + sid

    zeros16 = jnp.zeros((16,), jnp.float32)

    def zbody(k, c):
      bufs[0][k // grp, pl.ds((k % grp) * 16, 16)] = zeros16
      return c

    lax.fori_loop(0, CH * grp, zbody, 0)
    rbase = sid * RPT
    for k in range(RPT // CH):
      pltpu.sync_copy(bufs[0], acc.at[pl.ds(rbase + k * CH, CH)])
    plsc.subcore_barrier()

    def group(jj, c):
      j0 = jj * nbuf
      gcps = [
          pltpu.async_copy(h_hbm.at[sidx.at[j0 + b]], bufs[b], gsems[b])
          for b in range(nbuf)
      ]
      for b in range(nbuf):
        gcps[b].wait()
        pltpu.sync_copy(bufs[b], acc.at[didx.at[j0 + b]], add=True)
      return c

    for s in range(NSEC):
      pltpu.sync_copy(src_hbm.at[wid * NSEC + s], sidx)
      pltpu.sync_copy(dst_hbm.at[wid * NSEC + s], didx)
      lax.fori_loop(0, SEC // nbuf, group, 0)
    plsc.subcore_barrier()

    @pl.when(cid == 0)
    def _():
      pltpu.sync_copy(acc.at[pl.ds(rbase, RPT)], out0.at[pl.ds(rbase, RPT)])

    @pl.when(cid == 1)
    def _():
      pltpu.sync_copy(acc.at[pl.ds(rbase, RPT)], out1.at[pl.ds(rbase, RPT)])

  return agg


_AGG64 = _make_agg(H, 5)

_R = 1024
_NB = NPAD // _R


def _mlp_math(y_ref, p0_ref, p1_ref, aux_ref, w2_ref):
  aux = aux_ref[...]
  b1 = aux[0:1]
  gg = aux[1:2]
  bt = aux[2:3]
  mm = aux[3:4]
  vv = aux[4:5]
  b2 = aux[5:6]
  z = y_ref[...] + p0_ref[...] + p1_ref[...] + b1
  sc = gg * lax.rsqrt(vv + 1e-5)
  z = jnp.maximum(z * sc + (bt - mm * sc), 0.0)
  return jnp.maximum(
      jnp.dot(z, w2_ref[...], preferred_element_type=jnp.float32) + b2, 0.0)


def _pre_block(x_ref, w1_ref, o_ref):
  o_ref[...] = jnp.dot(x_ref[...], w1_ref[...],
                       preferred_element_type=jnp.float32)


def _pre_layer(x, w1):
  din = x.shape[1]
  return pl.pallas_call(
      _pre_block,
      grid=(_NB,),
      in_specs=[
          pl.BlockSpec((_R, din), lambda i: (i, 0)),
          pl.BlockSpec((din, H), lambda i: (0, 0)),
      ],
      out_specs=pl.BlockSpec((_R, H), lambda i: (i, 0)),
      out_shape=jax.ShapeDtypeStruct((NPAD, H), jnp.float32),
  )(x, w1)


def _mid_block(y_ref, p0_ref, p1_ref, aux_ref, w2_ref, wn_ref, o_ref):
  h = _mlp_math(y_ref, p0_ref, p1_ref, aux_ref, w2_ref)
  o_ref[...] = jnp.dot(h, wn_ref[...], preferred_element_type=jnp.float32)


def _mid_layer(y, p0, p1, aux, w2, wnext):
  return pl.pallas_call(
      _mid_block,
      grid=(_NB,),
      in_specs=[
          pl.BlockSpec((_R, H), lambda i: (i, 0)),
          pl.BlockSpec((_R, H), lambda i: (i, 0)),
          pl.BlockSpec((_R, H), lambda i: (i, 0)),
          pl.BlockSpec((8, H), lambda i: (0, 0)),
          pl.BlockSpec((H, H), lambda i: (0, 0)),
          pl.BlockSpec((H, H), lambda i: (0, 0)),
      ],
      out_specs=pl.BlockSpec((_R, H), lambda i: (i, 0)),
      out_shape=jax.ShapeDtypeStruct((NPAD, H), jnp.float32),
  )(y, p0, p1, aux, w2, wnext)


def _final_block(y_ref, p0_ref, p1_ref, aux_ref, w2_ref, bc_ref,
                 l1w_ref, l2w_ref, hb_ref, o_ref, pacc):
  i = pl.program_id(0)
  h3 = _mlp_math(y_ref, p0_ref, p1_ref, aux_ref, w2_ref)
  bc = bc_ref[...]
  seg = lax.broadcasted_iota(jnp.int32, (_R, G), 1)
  oh = (bc == seg).astype(jnp.float32)
  part = lax.dot_general(oh, h3, (((0,), (0,)), ((), ())),
                         preferred_element_type=jnp.float32)

  @pl.when(i == 0)
  def _():
    pacc[...] = part

  @pl.when(i > 0)
  def _():
    pacc[...] += part

  @pl.when(i == _NB - 1)
  def _():
    pooled = pacc[...]
    l1 = l1w_ref[...]
    leff = l1[0:H] + l1[H:2 * H] + l1[2 * H:3 * H]
    hb = hb_ref[...]
    h1 = jnp.maximum(
        jnp.dot(pooled, leff, preferred_element_type=jnp.float32) + hb[0:1, :],
        0.0)
    logits = jnp.dot(h1, l2w_ref[...],
                     preferred_element_type=jnp.float32) + hb[1:2, 0:2]
    mx = jnp.max(logits, axis=1, keepdims=True)
    s = jnp.sum(jnp.exp(logits - mx), axis=1, keepdims=True)
    o_ref[...] = logits - mx - jnp.log(s)


def _final_layer(y, p0, p1, aux, w2, bcol, l1w, l2w, hb):
  return pl.pallas_call(
      _final_block,
      grid=(_NB,),
      in_specs=[
          pl.BlockSpec((_R, H), lambda i: (i, 0)),
          pl.BlockSpec((_R, H), lambda i: (i, 0)),
          pl.BlockSpec((_R, H), lambda i: (i, 0)),
          pl.BlockSpec((8, H), lambda i: (0, 0)),
          pl.BlockSpec((H, H), lambda i: (0, 0)),
          pl.BlockSpec((_R, 1), lambda i: (i, 0)),
          pl.BlockSpec((3 * H, H), lambda i: (0, 0)),
          pl.BlockSpec((H, 2), lambda i: (0, 0)),
          pl.BlockSpec((8, H), lambda i: (0, 0)),
      ],
      out_specs=pl.BlockSpec((G, 2), lambda i: (0, 0)),
      out_shape=jax.ShapeDtypeStruct((G, 2), jnp.float32),
      scratch_shapes=[pltpu.VMEM((G, H), jnp.float32)],
  )(y, p0, p1, aux, w2, bcol, l1w, l2w, hb)


def _aux_stack(b1, g, bt, m, v, b2):
  return jnp.concatenate(
      [b1[None], g[None], bt[None], m[None], v[None], b2[None],
       jnp.zeros((2, H), jnp.float32)], axis=0)


def kernel(x, edge_index, batch, W1_0, b1_0, g_0, bt_0, m_0, v_0, W2_0, b2_0,
           W1_1, b1_1, g_1, bt_1, m_1, v_1, W2_1, b2_1,
           W1_2, b1_2, g_2, bt_2, m_2, v_2, W2_2, b2_2,
           lin1_W, lin1_b, lin2_W, lin2_b):
  pe = EPAD - E
  srcp = jnp.concatenate(
      [edge_index[0],
       jnp.arange(pe, dtype=jnp.int32) % N]).reshape(NW * NSEC, SEC, CH)
  dstp = jnp.concatenate(
      [edge_index[1],
       N + (jnp.arange(pe, dtype=jnp.int32) % (NPAD - N))]).reshape(
           NW * NSEC, SEC, CH)
  xp = jnp.concatenate([x, jnp.zeros((NPAD - N, D), jnp.float32)], axis=0)
  bcol = jnp.concatenate(
      [batch, jnp.full((NPAD - N,), G, jnp.int32)]).reshape(NPAD, 1)
  aux0 = _aux_stack(b1_0, g_0, bt_0, m_0, v_0, b2_0)
  aux1 = _aux_stack(b1_1, g_1, bt_1, m_1, v_1, b2_1)
  aux2 = _aux_stack(b1_2, g_2, bt_2, m_2, v_2, b2_2)
  hb = jnp.zeros((8, H), jnp.float32).at[0].set(lin1_b).at[1, :2].set(lin2_b)

  y0 = _pre_layer(xp, W1_0)
  p0, p1 = _AGG64(y0, srcp, dstp)
  y1 = _mid_layer(y0, p0, p1, aux0, W2_0, W1_1)
  p0, p1 = _AGG64(y1, srcp, dstp)
  y2 = _mid_layer(y1, p0, p1, aux1, W2_1, W1_2)
  p0, p1 = _AGG64(y2, srcp, dstp)
  return _final_layer(y2, p0, p1, aux2, W2_2, bcol, lin1_W, lin2_W, hb)


# speedup vs baseline: 4.3323x; 1.2844x over previous
import functools

import jax
import jax.numpy as jnp
from jax import lax
from jax.experimental import pallas as pl
from jax.experimental.pallas import tpu as pltpu
from jax.experimental.pallas import tpu_sc as plsc

N = 10000
NPAD = 10240
E = 320000
D = 128
H = 64
G = 64
NC = 2
NS = 16
NW = NC * NS
CH = 128
NCHUNK = 80
EPT = NCHUNK * CH
EPAD = NW * EPT
RPT = NPAD // NS
GRP = 2
NSET = 4
NG = NCHUNK // GRP


def _make_agg(dh):
  grp16 = dh // 16
  mesh = plsc.VectorSubcoreMesh(core_axis_name="c", subcore_axis_name="s")

  @functools.partial(
      pl.kernel,
      mesh=mesh,
      compiler_params=pltpu.CompilerParams(use_tc_tiling_on_sc=False),
      out_type=[
          jax.ShapeDtypeStruct((NPAD, dh), jnp.float32),
          jax.ShapeDtypeStruct((NPAD, dh), jnp.float32),
      ],
      scratch_types=[
          pltpu.VMEM((NCHUNK, CH), jnp.int32),
          pltpu.VMEM((NCHUNK, CH), jnp.int32),
          [pltpu.VMEM((CH, dh), jnp.float32) for _ in range(NSET * GRP)],
          pltpu.VMEM_SHARED((NPAD, dh), jnp.float32),
          [pltpu.SemaphoreType.DMA for _ in range(NSET)],
          [pltpu.SemaphoreType.DMA for _ in range(NSET)],
      ],
  )
  def agg(h_hbm, src_hbm, dst_hbm, out0, out1, sidx, didx, bufs, acc,
          gsems, ssems):
    cid = lax.axis_index("c")
    sid = lax.axis_index("s")
    wid = cid * NS + sid

    zeros16 = jnp.zeros((16,), jnp.float32)

    def zbody(k, c):
      bufs[0][k // grp16, pl.ds((k % grp16) * 16, 16)] = zeros16
      return c

    lax.fori_loop(0, CH * grp16, zbody, 0)
    rbase = sid * RPT
    for k in range(RPT // CH):
      pltpu.sync_copy(bufs[0], acc.at[pl.ds(rbase + k * CH, CH)])
    plsc.subcore_barrier()

    pltpu.sync_copy(src_hbm.at[wid], sidx)
    pltpu.sync_copy(dst_hbm.at[wid], didx)

    def fire_g(s, g):
      for b in range(GRP):
        pltpu.async_copy(h_hbm.at[sidx.at[g * GRP + b]], bufs[s * GRP + b],
                         gsems[s])

    def wait_g(s, g):
      for b in range(GRP):
        pltpu.make_async_copy(h_hbm.at[sidx.at[g * GRP + b]],
                              bufs[s * GRP + b], gsems[s]).wait()

    def fire_s(s, g):
      for b in range(GRP):
        pltpu.async_copy(bufs[s * GRP + b], acc.at[didx.at[g * GRP + b]],
                         ssems[s], add=True)

    def wait_s(s, g):
      for b in range(GRP):
        pltpu.make_async_copy(bufs[s * GRP + b],
                              acc.at[didx.at[g * GRP + b]], ssems[s]).wait()

    def one(s, g):
      wait_g(s, g)
      fire_s(s, g)
      wait_s((s + NSET - 1) % NSET, g - 1)
      fire_g((s + NSET - 1) % NSET, g + NSET - 1)

    for g0 in range(NSET - 1):
      fire_g(g0, g0)
    wait_g(0, 0)
    fire_s(0, 0)
    fire_g(NSET - 1, NSET - 1)

    def quad(p, c):
      gb = NSET * p + 1
      for k in range(NSET):
        one((1 + k) % NSET, gb + k)
      return c

    lax.fori_loop(0, (NG - NSET) // NSET, quad, 0)

    for g in range(NG - NSET + 1, NG):
      s = g % NSET
      wait_g(s, g)
      fire_s(s, g)
      wait_s((s + NSET - 1) % NSET, g - 1)
    wait_s((NG - 1) % NSET, NG - 1)
    plsc.subcore_barrier()

    @pl.when(cid == 0)
    def _():
      pltpu.sync_copy(acc.at[pl.ds(rbase, RPT)], out0.at[pl.ds(rbase, RPT)])

    @pl.when(cid == 1)
    def _():
      pltpu.sync_copy(acc.at[pl.ds(rbase, RPT)], out1.at[pl.ds(rbase, RPT)])

  return agg


_AGG64 = _make_agg(H)

_R = 1024
_NB = NPAD // _R


def _mlp_math(y_ref, p0_ref, p1_ref, aux_ref, w2_ref):
  aux = aux_ref[...]
  b1 = aux[0:1]
  gg = aux[1:2]
  bt = aux[2:3]
  mm = aux[3:4]
  vv = aux[4:5]
  b2 = aux[5:6]
  z = y_ref[...] + p0_ref[...] + p1_ref[...] + b1
  sc = gg * lax.rsqrt(vv + 1e-5)
  z = jnp.maximum(z * sc + (bt - mm * sc), 0.0)
  return jnp.maximum(
      jnp.dot(z, w2_ref[...], preferred_element_type=jnp.float32) + b2, 0.0)


def _pre_block(x_ref, w1_ref, o_ref):
  o_ref[...] = jnp.dot(x_ref[...], w1_ref[...],
                       preferred_element_type=jnp.float32)


def _pre_layer(x, w1):
  din = x.shape[1]
  return pl.pallas_call(
      _pre_block,
      grid=(_NB,),
      in_specs=[
          pl.BlockSpec((_R, din), lambda i: (i, 0)),
          pl.BlockSpec((din, H), lambda i: (0, 0)),
      ],
      out_specs=pl.BlockSpec((_R, H), lambda i: (i, 0)),
      out_shape=jax.ShapeDtypeStruct((NPAD, H), jnp.float32),
  )(x, w1)


def _mid_block(y_ref, p0_ref, p1_ref, aux_ref, w2_ref, wn_ref, o_ref):
  h = _mlp_math(y_ref, p0_ref, p1_ref, aux_ref, w2_ref)
  o_ref[...] = jnp.dot(h, wn_ref[...], preferred_element_type=jnp.float32)


def _mid_layer(y, p0, p1, aux, w2, wnext):
  return pl.pallas_call(
      _mid_block,
      grid=(_NB,),
      in_specs=[
          pl.BlockSpec((_R, H), lambda i: (i, 0)),
          pl.BlockSpec((_R, H), lambda i: (i, 0)),
          pl.BlockSpec((_R, H), lambda i: (i, 0)),
          pl.BlockSpec((8, H), lambda i: (0, 0)),
          pl.BlockSpec((H, H), lambda i: (0, 0)),
          pl.BlockSpec((H, H), lambda i: (0, 0)),
      ],
      out_specs=pl.BlockSpec((_R, H), lambda i: (i, 0)),
      out_shape=jax.ShapeDtypeStruct((NPAD, H), jnp.float32),
  )(y, p0, p1, aux, w2, wnext)


def _final_block(y_ref, p0_ref, p1_ref, aux_ref, w2_ref, bc_ref,
                 l1w_ref, l2w_ref, hb_ref, o_ref, pacc):
  i = pl.program_id(0)
  h3 = _mlp_math(y_ref, p0_ref, p1_ref, aux_ref, w2_ref)
  bc = bc_ref[...]
  seg = lax.broadcasted_iota(jnp.int32, (_R, G), 1)
  oh = (bc == seg).astype(jnp.float32)
  part = lax.dot_general(oh, h3, (((0,), (0,)), ((), ())),
                         preferred_element_type=jnp.float32)

  @pl.when(i == 0)
  def _():
    pacc[...] = part

  @pl.when(i > 0)
  def _():
    pacc[...] += part

  @pl.when(i == _NB - 1)
  def _():
    pooled = pacc[...]
    l1 = l1w_ref[...]
    leff = l1[0:H] + l1[H:2 * H] + l1[2 * H:3 * H]
    hb = hb_ref[...]
    h1 = jnp.maximum(
        jnp.dot(pooled, leff, preferred_element_type=jnp.float32) + hb[0:1, :],
        0.0)
    logits = jnp.dot(h1, l2w_ref[...],
                     preferred_element_type=jnp.float32) + hb[1:2, 0:2]
    mx = jnp.max(logits, axis=1, keepdims=True)
    s = jnp.sum(jnp.exp(logits - mx), axis=1, keepdims=True)
    o_ref[...] = logits - mx - jnp.log(s)


def _final_layer(y, p0, p1, aux, w2, bcol, l1w, l2w, hb):
  return pl.pallas_call(
      _final_block,
      grid=(_NB,),
      in_specs=[
          pl.BlockSpec((_R, H), lambda i: (i, 0)),
          pl.BlockSpec((_R, H), lambda i: (i, 0)),
          pl.BlockSpec((_R, H), lambda i: (i, 0)),
          pl.BlockSpec((8, H), lambda i: (0, 0)),
          pl.BlockSpec((H, H), lambda i: (0, 0)),
          pl.BlockSpec((_R, 1), lambda i: (i, 0)),
          pl.BlockSpec((3 * H, H), lambda i: (0, 0)),
          pl.BlockSpec((H, 2), lambda i: (0, 0)),
          pl.BlockSpec((8, H), lambda i: (0, 0)),
      ],
      out_specs=pl.BlockSpec((G, 2), lambda i: (0, 0)),
      out_shape=jax.ShapeDtypeStruct((G, 2), jnp.float32),
      scratch_shapes=[pltpu.VMEM((G, H), jnp.float32)],
  )(y, p0, p1, aux, w2, bcol, l1w, l2w, hb)


def _aux_stack(b1, g, bt, m, v, b2):
  return jnp.concatenate(
      [b1[None], g[None], bt[None], m[None], v[None], b2[None],
       jnp.zeros((2, H), jnp.float32)], axis=0)


def kernel(x, edge_index, batch, W1_0, b1_0, g_0, bt_0, m_0, v_0, W2_0, b2_0,
           W1_1, b1_1, g_1, bt_1, m_1, v_1, W2_1, b2_1,
           W1_2, b1_2, g_2, bt_2, m_2, v_2, W2_2, b2_2,
           lin1_W, lin1_b, lin2_W, lin2_b):
  pe = EPAD - E
  srcp = jnp.concatenate(
      [edge_index[0],
       jnp.arange(pe, dtype=jnp.int32) % N]).reshape(NW, NCHUNK, CH)
  dstp = jnp.concatenate(
      [edge_index[1],
       N + (jnp.arange(pe, dtype=jnp.int32) % (NPAD - N))]).reshape(
           NW, NCHUNK, CH)
  xp = jnp.concatenate([x, jnp.zeros((NPAD - N, D), jnp.float32)], axis=0)
  bcol = jnp.concatenate(
      [batch, jnp.full((NPAD - N,), G, jnp.int32)]).reshape(NPAD, 1)
  aux0 = _aux_stack(b1_0, g_0, bt_0, m_0, v_0, b2_0)
  aux1 = _aux_stack(b1_1, g_1, bt_1, m_1, v_1, b2_1)
  aux2 = _aux_stack(b1_2, g_2, bt_2, m_2, v_2, b2_2)
  hb = jnp.zeros((8, H), jnp.float32).at[0].set(lin1_b).at[1, :2].set(lin2_b)

  y0 = _pre_layer(xp, W1_0)
  p0, p1 = _AGG64(y0, srcp, dstp)
  y1 = _mid_layer(y0, p0, p1, aux0, W2_0, W1_1)
  p0, p1 = _AGG64(y1, srcp, dstp)
  y2 = _mid_layer(y1, p0, p1, aux1, W2_1, W1_2)
  p0, p1 = _AGG64(y2, srcp, dstp)
  return _final_layer(y2, p0, p1, aux2, W2_2, bcol, lin1_W, lin2_W, hb)


# speedup vs baseline: 4.4311x; 1.0228x over previous
import functools

import jax
import jax.numpy as jnp
from jax import lax
from jax.experimental import pallas as pl
from jax.experimental.pallas import tpu as pltpu
from jax.experimental.pallas import tpu_sc as plsc

N = 10000
NPAD = 10240
E = 320000
D = 128
H = 64
G = 64
NC = 2
NS = 16
NW = NC * NS
CH = 128
NCHUNK = 80
EPT = NCHUNK * CH
EPAD = NW * EPT
RPT = NPAD // NS
GRP = 2
NSET = 4
NG = NCHUNK // GRP


def _make_agg(dh):
  grp16 = dh // 16
  mesh = plsc.VectorSubcoreMesh(core_axis_name="c", subcore_axis_name="s")

  @functools.partial(
      pl.kernel,
      mesh=mesh,
      compiler_params=pltpu.CompilerParams(use_tc_tiling_on_sc=False),
      out_type=[
          jax.ShapeDtypeStruct((NPAD, dh), jnp.float32),
          jax.ShapeDtypeStruct((NPAD, dh), jnp.float32),
      ],
      scratch_types=[
          pltpu.VMEM((NCHUNK, CH), jnp.int32),
          pltpu.VMEM((NCHUNK, CH), jnp.int32),
          [pltpu.VMEM((CH, dh), jnp.float32) for _ in range(NSET * GRP)],
          pltpu.VMEM_SHARED((NPAD, dh), jnp.float32),
          [pltpu.SemaphoreType.DMA for _ in range(NSET)],
          [pltpu.SemaphoreType.DMA for _ in range(NSET)],
      ],
  )
  def agg(h_hbm, src_hbm, dst_hbm, out0, out1, sidx, didx, bufs, acc,
          gsems, ssems):
    cid = lax.axis_index("c")
    sid = lax.axis_index("s")
    wid = cid * NS + sid

    def fire_g(s, g):
      for b in range(GRP):
        pltpu.async_copy(h_hbm.at[sidx.at[g * GRP + b]], bufs[s * GRP + b],
                         gsems[s])

    def wait_g(s, g):
      for b in range(GRP):
        pltpu.make_async_copy(h_hbm.at[sidx.at[g * GRP + b]],
                              bufs[s * GRP + b], gsems[s]).wait()

    def fire_s(s, g):
      for b in range(GRP):
        pltpu.async_copy(bufs[s * GRP + b], acc.at[didx.at[g * GRP + b]],
                         ssems[s], add=True)

    def wait_s(s, g):
      for b in range(GRP):
        pltpu.make_async_copy(bufs[s * GRP + b],
                              acc.at[didx.at[g * GRP + b]], ssems[s]).wait()

    def one(s, g):
      wait_g(s, g)
      fire_s(s, g)
      wait_s((s + NSET - 1) % NSET, g - 1)
      fire_g((s + NSET - 1) % NSET, g + NSET - 1)

    pltpu.sync_copy(src_hbm.at[wid], sidx)
    pltpu.sync_copy(dst_hbm.at[wid], didx)
    for g0 in range(NSET - 1):
      fire_g(g0, g0)

    zeros16 = jnp.zeros((16,), jnp.float32)
    zbuf = bufs[NSET * GRP - 1]

    def zbody(k, c):
      zbuf[k // grp16, pl.ds((k % grp16) * 16, 16)] = zeros16
      return c

    lax.fori_loop(0, CH * grp16, zbody, 0)
    rbase = sid * RPT
    for k in range(RPT // CH):
      pltpu.sync_copy(zbuf, acc.at[pl.ds(rbase + k * CH, CH)])
    plsc.subcore_barrier()

    wait_g(0, 0)
    fire_s(0, 0)
    fire_g(NSET - 1, NSET - 1)

    def quad(p, c):
      gb = NSET * p + 1
      for k in range(NSET):
        one((1 + k) % NSET, gb + k)
      return c

    lax.fori_loop(0, (NG - NSET) // NSET, quad, 0)

    for g in range(NG - NSET + 1, NG):
      s = g % NSET
      wait_g(s, g)
      fire_s(s, g)
      wait_s((s + NSET - 1) % NSET, g - 1)
    wait_s((NG - 1) % NSET, NG - 1)
    plsc.subcore_barrier()

    @pl.when(cid == 0)
    def _():
      pltpu.sync_copy(acc.at[pl.ds(rbase, RPT)], out0.at[pl.ds(rbase, RPT)])

    @pl.when(cid == 1)
    def _():
      pltpu.sync_copy(acc.at[pl.ds(rbase, RPT)], out1.at[pl.ds(rbase, RPT)])

  return agg


_AGG64 = _make_agg(H)

_R = 1024
_NB = NPAD // _R


def _mlp_math(y_ref, p0_ref, p1_ref, aux_ref, w2_ref):
  aux = aux_ref[...]
  b1 = aux[0:1]
  gg = aux[1:2]
  bt = aux[2:3]
  mm = aux[3:4]
  vv = aux[4:5]
  b2 = aux[5:6]
  z = y_ref[...] + p0_ref[...] + p1_ref[...] + b1
  sc = gg * lax.rsqrt(vv + 1e-5)
  z = jnp.maximum(z * sc + (bt - mm * sc), 0.0)
  return jnp.maximum(
      jnp.dot(z, w2_ref[...], preferred_element_type=jnp.float32) + b2, 0.0)


def _pre_block(x_ref, w1_ref, o_ref):
  o_ref[...] = jnp.dot(x_ref[...], w1_ref[...],
                       preferred_element_type=jnp.float32)


def _pre_layer(x, w1):
  din = x.shape[1]
  return pl.pallas_call(
      _pre_block,
      grid=(_NB,),
      in_specs=[
          pl.BlockSpec((_R, din), lambda i: (i, 0)),
          pl.BlockSpec((din, H), lambda i: (0, 0)),
      ],
      out_specs=pl.BlockSpec((_R, H), lambda i: (i, 0)),
      out_shape=jax.ShapeDtypeStruct((NPAD, H), jnp.float32),
  )(x, w1)


def _mid_block(y_ref, p0_ref, p1_ref, aux_ref, w2_ref, wn_ref, o_ref):
  h = _mlp_math(y_ref, p0_ref, p1_ref, aux_ref, w2_ref)
  o_ref[...] = jnp.dot(h, wn_ref[...], preferred_element_type=jnp.float32)


def _mid_layer(y, p0, p1, aux, w2, wnext):
  return pl.pallas_call(
      _mid_block,
      grid=(_NB,),
      in_specs=[
          pl.BlockSpec((_R, H), lambda i: (i, 0)),
          pl.BlockSpec((_R, H), lambda i: (i, 0)),
          pl.BlockSpec((_R, H), lambda i: (i, 0)),
          pl.BlockSpec((8, H), lambda i: (0, 0)),
          pl.BlockSpec((H, H), lambda i: (0, 0)),
          pl.BlockSpec((H, H), lambda i: (0, 0)),
      ],
      out_specs=pl.BlockSpec((_R, H), lambda i: (i, 0)),
      out_shape=jax.ShapeDtypeStruct((NPAD, H), jnp.float32),
  )(y, p0, p1, aux, w2, wnext)


def _final_block(y_ref, p0_ref, p1_ref, aux_ref, w2_ref, bc_ref,
                 l1w_ref, l2w_ref, hb_ref, o_ref, pacc):
  i = pl.program_id(0)
  h3 = _mlp_math(y_ref, p0_ref, p1_ref, aux_ref, w2_ref)
  bc = bc_ref[...]
  seg = lax.broadcasted_iota(jnp.int32, (_R, G), 1)
  oh = (bc == seg).astype(jnp.float32)
  part = lax.dot_general(oh, h3, (((0,), (0,)), ((), ())),
                         preferred_element_type=jnp.float32)

  @pl.when(i == 0)
  def _():
    pacc[...] = part

  @pl.when(i > 0)
  def _():
    pacc[...] += part

  @pl.when(i == _NB - 1)
  def _():
    pooled = pacc[...]
    l1 = l1w_ref[...]
    leff = l1[0:H] + l1[H:2 * H] + l1[2 * H:3 * H]
    hb = hb_ref[...]
    h1 = jnp.maximum(
        jnp.dot(pooled, leff, preferred_element_type=jnp.float32) + hb[0:1, :],
        0.0)
    logits = jnp.dot(h1, l2w_ref[...],
                     preferred_element_type=jnp.float32) + hb[1:2, 0:2]
    mx = jnp.max(logits, axis=1, keepdims=True)
    s = jnp.sum(jnp.exp(logits - mx), axis=1, keepdims=True)
    o_ref[...] = logits - mx - jnp.log(s)


def _final_layer(y, p0, p1, aux, w2, bcol, l1w, l2w, hb):
  return pl.pallas_call(
      _final_block,
      grid=(_NB,),
      in_specs=[
          pl.BlockSpec((_R, H), lambda i: (i, 0)),
          pl.BlockSpec((_R, H), lambda i: (i, 0)),
          pl.BlockSpec((_R, H), lambda i: (i, 0)),
          pl.BlockSpec((8, H), lambda i: (0, 0)),
          pl.BlockSpec((H, H), lambda i: (0, 0)),
          pl.BlockSpec((_R, 1), lambda i: (i, 0)),
          pl.BlockSpec((3 * H, H), lambda i: (0, 0)),
          pl.BlockSpec((H, 2), lambda i: (0, 0)),
          pl.BlockSpec((8, H), lambda i: (0, 0)),
      ],
      out_specs=pl.BlockSpec((G, 2), lambda i: (0, 0)),
      out_shape=jax.ShapeDtypeStruct((G, 2), jnp.float32),
      scratch_shapes=[pltpu.VMEM((G, H), jnp.float32)],
  )(y, p0, p1, aux, w2, bcol, l1w, l2w, hb)


def _aux_stack(b1, g, bt, m, v, b2):
  return jnp.concatenate(
      [b1[None], g[None], bt[None], m[None], v[None], b2[None],
       jnp.zeros((2, H), jnp.float32)], axis=0)


def kernel(x, edge_index, batch, W1_0, b1_0, g_0, bt_0, m_0, v_0, W2_0, b2_0,
           W1_1, b1_1, g_1, bt_1, m_1, v_1, W2_1, b2_1,
           W1_2, b1_2, g_2, bt_2, m_2, v_2, W2_2, b2_2,
           lin1_W, lin1_b, lin2_W, lin2_b):
  pe = EPAD - E
  srcp = jnp.concatenate(
      [edge_index[0],
       jnp.arange(pe, dtype=jnp.int32) % N]).reshape(NW, NCHUNK, CH)
  dstp = jnp.concatenate(
      [edge_index[1],
       N + (jnp.arange(pe, dtype=jnp.int32) % (NPAD - N))]).reshape(
           NW, NCHUNK, CH)
  xp = jnp.concatenate([x, jnp.zeros((NPAD - N, D), jnp.float32)], axis=0)
  bcol = jnp.concatenate(
      [batch, jnp.full((NPAD - N,), G, jnp.int32)]).reshape(NPAD, 1)
  aux0 = _aux_stack(b1_0, g_0, bt_0, m_0, v_0, b2_0)
  aux1 = _aux_stack(b1_1, g_1, bt_1, m_1, v_1, b2_1)
  aux2 = _aux_stack(b1_2, g_2, bt_2, m_2, v_2, b2_2)
  hb = jnp.zeros((8, H), jnp.float32).at[0].set(lin1_b).at[1, :2].set(lin2_b)

  y0 = _pre_layer(xp, W1_0)
  p0, p1 = _AGG64(y0, srcp, dstp)
  y1 = _mid_layer(y0, p0, p1, aux0, W2_0, W1_1)
  p0, p1 = _AGG64(y1, srcp, dstp)
  y2 = _mid_layer(y1, p0, p1, aux1, W2_1, W1_2)
  p0, p1 = _AGG64(y2, srcp, dstp)
  return _final_layer(y2, p0, p1, aux2, W2_2, bcol, lin1_W, lin2_W, hb)
